# baseline (speedup 1.0000x reference)
import functools
import jax
import jax.numpy as jnp
from jax import lax
from jax.experimental import pallas as pl
from jax.experimental.pallas import tpu as pltpu
from jax.experimental.pallas import tpu_sc as plsc

_TAPS8 = [(s0, s1, s2) for s0 in (0, 1) for s1 in (0, 1) for s2 in (0, 1)]
_BF = jnp.bfloat16


def _s2d(xp):
    n, d, h, w, c = xp.shape
    x = xp.reshape(n, d // 2, 2, h // 2, 2, w // 2, 2, c)
    x = x.transpose(0, 1, 3, 5, 2, 4, 6, 7)
    return x.reshape(n, d // 2, h // 2, w // 2, 8 * c)


def _enc_w(w):
    o, c = w.shape[:2]
    wr = w.astype(_BF).reshape(o, c, 2, 2, 2, 2, 2, 2)
    wr = wr.transpose(2, 4, 6, 3, 5, 7, 1, 0)
    return wr.reshape(8, 8 * c, o)


def _dec_w(w):
    c, o = w.shape[:2]
    wf = w.reshape(c, o, 2, 2, 2, 2, 2, 2)
    wr = wf.transpose(3, 5, 7, 2, 4, 6, 0, 1)
    return wr.reshape(8, 8, c, o)


def _win8(x, o):
    views = [x[:, s0:s0 + o, s1:s1 + o, s2:s2 + o, :] for s0, s1, s2 in _TAPS8]
    return jnp.stack(views, axis=4)


def _interleave(y, n, g, c):
    y = y.reshape(2, 2, 2, n, g, g, g, c)
    y = y.transpose(3, 4, 0, 5, 1, 6, 2, 7)
    return y.reshape(n, 2 * g, 2 * g, 2 * g, c)


def _conv1_body(x_ref, w_ref, b_ref, o_ref, *, g):
    x = x_ref[0]
    c = x.shape[-1]
    acc = None
    for t, (s0, s1, s2) in enumerate(_TAPS8):
        a = x[s0:s0 + g, s1:s1 + g, s2:s2 + g, :].reshape(g ** 3, c)
        p = jnp.dot(a, w_ref[t], preferred_element_type=jnp.float32)
        acc = p if acc is None else acc + p
    o_ref[0] = jnp.maximum(acc + b_ref[...], 0.0).astype(o_ref.dtype)


def _conv1(x, w, b2d, out_dtype):
    n, gp = x.shape[0], x.shape[1]
    g = gp - 1
    c = x.shape[-1]
    o = w.shape[2]
    return pl.pallas_call(
        functools.partial(_conv1_body, g=g),
        grid=(n,),
        in_specs=[
            pl.BlockSpec((1, gp, gp, gp, c), lambda i: (i, 0, 0, 0, 0)),
            pl.BlockSpec((8, c, o), lambda i: (0, 0, 0)),
            pl.BlockSpec((1, o), lambda i: (0, 0)),
        ],
        out_specs=pl.BlockSpec((1, g ** 3, o), lambda i: (i, 0, 0)),
        out_shape=jax.ShapeDtypeStruct((n, g ** 3, o), out_dtype),
    )(x, w, b2d)


def _sconv_body(x_ref, w_ref, b_ref, o_ref, *, g, relu):
    x = x_ref[0]
    c = x.shape[-1]
    acc = None
    for t, (s0, s1, s2) in enumerate(_TAPS8):
        a = x[s0:s0 + g, s1:s1 + g, s2:s2 + g, :].reshape(g ** 3, c)
        p = jnp.dot(a, w_ref[t], preferred_element_type=jnp.float32)
        acc = p if acc is None else acc + p
    acc = acc + b_ref[...]
    if relu:
        acc = jnp.maximum(acc, 0.0)
    o_ref[...] = acc.astype(o_ref.dtype)


def _sconv(x, w, b2d, relu, out_dtype):
    n, gp = x.shape[0], x.shape[1]
    g = gp - 1
    c, o = w.shape[1], w.shape[2]
    return pl.pallas_call(
        functools.partial(_sconv_body, g=g, relu=relu),
        grid=(n,),
        in_specs=[
            pl.BlockSpec((1, gp, gp, gp, c), lambda i: (i, 0, 0, 0, 0)),
            pl.BlockSpec((8, c, o), lambda i: (0, 0, 0)),
            pl.BlockSpec((1, o), lambda i: (0, 0)),
        ],
        out_specs=pl.BlockSpec((g ** 3, o), lambda i: (i, 0)),
        out_shape=jax.ShapeDtypeStruct((n * g ** 3, o), out_dtype),
    )(x, w, b2d)


def _dconvp_body(x_ref, w_ref, b_ref, o_ref, *, g, relu):
    x = x_ref[0]
    c = x.shape[-1]
    for r, (r0, r1, r2) in enumerate(_TAPS8):
        acc = None
        for t, (s0, s1, s2) in enumerate(_TAPS8):
            a = x[r0 + s0:r0 + s0 + g, r1 + s1:r1 + s1 + g,
                  r2 + s2:r2 + s2 + g, :].reshape(g ** 3, c)
            p = jnp.dot(a, w_ref[7 - r, 7 - t],
                        preferred_element_type=jnp.float32)
            acc = p if acc is None else acc + p
        acc = acc + b_ref[...]
        if relu:
            acc = jnp.maximum(acc, 0.0)
        o_ref[r] = acc.astype(o_ref.dtype)


def _dconvp(x, w, b2d, relu, out_dtype):
    n, gp = x.shape[0], x.shape[1]
    g = gp - 2
    c, o = w.shape[2], w.shape[3]
    return pl.pallas_call(
        functools.partial(_dconvp_body, g=g, relu=relu),
        grid=(n,),
        in_specs=[
            pl.BlockSpec((1, gp, gp, gp, c), lambda i: (i, 0, 0, 0, 0)),
            pl.BlockSpec((8, 8, c, o), lambda i: (0, 0, 0, 0)),
            pl.BlockSpec((1, o), lambda i: (0, 0)),
        ],
        out_specs=pl.BlockSpec((8, g ** 3, o), lambda i: (0, i, 0)),
        out_shape=jax.ShapeDtypeStruct((8, n * g ** 3, o), out_dtype),
    )(x, w, b2d)


def _dconv3_body(x_ref, w_ref, b_ref, o_ref, *, g):
    x = x_ref[0]
    c = x.shape[-1]
    acc = None
    i = 0
    for a0 in range(3):
        for a1 in range(3):
            for a2 in range(3):
                a = x[a0:a0 + g, a1:a1 + g, a2:a2 + g, :].reshape(g ** 3, c)
                p = jnp.dot(a, w_ref[i], preferred_element_type=jnp.float32)
                acc = p if acc is None else acc + p
                i += 1
    o_ref[0] = acc + b_ref[...]


def _dconv3(x, w27, b2d):
    n, gp = x.shape[0], x.shape[1]
    g = gp - 2
    c = x.shape[-1]
    return pl.pallas_call(
        functools.partial(_dconv3_body, g=g),
        grid=(n,),
        in_specs=[
            pl.BlockSpec((1, gp, gp, gp, c), lambda i: (i, 0, 0, 0, 0)),
            pl.BlockSpec((27, c, 8), lambda i: (0, 0, 0)),
            pl.BlockSpec((1, 8), lambda i: (0, 0)),
        ],
        out_specs=pl.BlockSpec((1, g * g * g, 8), lambda i: (i, 0, 0)),
        out_shape=jax.ShapeDtypeStruct((n, g * g * g, 8), jnp.float32),
    )(x, w27, b2d)


def _vq_body(flat_ref, emb_ref, idx_ref, loss_ref):
    flat = flat_ref[...]
    emb = emb_ref[...]
    f2 = jnp.sum(flat * flat, axis=1, keepdims=True)
    e2 = jnp.sum(emb * emb, axis=1)
    xe = jnp.dot(flat, emb.T, preferred_element_type=jnp.float32)
    dist = f2 + e2[None, :] - 2.0 * xe
    minv = jnp.min(dist, axis=1, keepdims=True)
    k = emb.shape[0]
    iota = lax.broadcasted_iota(jnp.int32, dist.shape, 1)
    idx_ref[...] = jnp.min(jnp.where(dist <= minv, iota, k), axis=1)
    loss_ref[...] = jnp.reshape(
        1.25 * jnp.sum(jnp.maximum(minv, 0.0)) / flat.size, (1, 1))


def _vq_assign(flat, emb):
    m, _ = flat.shape
    idx, loss = pl.pallas_call(
        _vq_body,
        out_shape=[
            jax.ShapeDtypeStruct((m,), jnp.int32),
            jax.ShapeDtypeStruct((1, 1), jnp.float32),
        ],
    )(flat, emb)
    return idx, loss[0, 0]


def _sc_gather(emb, idx):
    info = plsc.get_sparse_core_info()
    nw = info.num_cores * info.num_subcores
    b = idx.shape[0]
    d = emb.shape[1]
    bpw = b // nw
    nc = info.num_cores
    mesh = plsc.VectorSubcoreMesh(core_axis_name="c", subcore_axis_name="s")

    @functools.partial(
        pl.kernel, mesh=mesh,
        out_type=jax.ShapeDtypeStruct((b, d), jnp.float32),
        scratch_types=[
            pltpu.VMEM((bpw,), jnp.int32),
            pltpu.VMEM((bpw, d), jnp.float32),
            pltpu.SemaphoreType.DMA,
        ],
    )
    def k(emb_hbm, idx_hbm, out_hbm, idx_v, rows_v, sem):
        wid = lax.axis_index("s") * nc + lax.axis_index("c")
        base = wid * bpw
        pltpu.sync_copy(idx_hbm.at[pl.ds(base, bpw)], idx_v)
        pltpu.async_copy(emb_hbm.at[idx_v], rows_v, sem).wait()
        pltpu.sync_copy(rows_v, out_hbm.at[pl.ds(base, bpw)])

    return k(emb, idx)


def kernel(x, emb, w1, b1, w2, b2, w3, b3, dw1, db1, dw2, db2, dw3, db3, wo, bo):
    n = x.shape[0]

    xl = x.reshape(n, 32, 32, 32, 1).astype(_BF)
    xp = jnp.pad(xl, ((0, 0), (1, 1), (1, 1), (1, 1), (0, 0)))
    x2 = _s2d(xp)
    h1 = _conv1(x2, _enc_w(w1), b1.reshape(1, 128), _BF)
    h1 = h1.reshape(n, 16, 16, 16, 128)

    h1p = jnp.pad(h1, ((0, 0), (1, 1), (1, 1), (1, 1), (0, 0)))
    h1s = _s2d(h1p)
    h2 = _sconv(h1s, _enc_w(w2), b2.reshape(1, 256), True, _BF)
    h2 = h2.reshape(n, 8, 8, 8, 256)

    h2p = jnp.pad(h2, ((0, 0), (1, 1), (1, 1), (1, 1), (0, 0)))
    h2s = _s2d(h2p)
    h3 = _sconv(h2s, _enc_w(w3), b3.reshape(1, 256), False, jnp.float32)

    flat = h3.reshape(-1, emb.shape[1])
    idx, eq_loss = _vq_assign(flat, emb)
    quant = _sc_gather(emb, idx)

    q5 = quant.astype(_BF).reshape(n, 4, 4, 4, 256)
    qp = jnp.pad(q5, ((0, 0), (1, 1), (1, 1), (1, 1), (0, 0)))
    y1 = _dconvp(qp, _dec_w(dw1).astype(_BF), db1.reshape(1, 256),
                 True, _BF)
    r1 = _interleave(y1, n, 4, 256)

    r1p = jnp.pad(r1, ((0, 0), (1, 1), (1, 1), (1, 1), (0, 0)))
    y2 = _dconvp(r1p, _dec_w(dw2).astype(_BF), db2.reshape(1, 128),
                 True, _BF)
    r2 = _interleave(y2, n, 8, 128)

    wov = wo[0, :, 0, 0, 0]
    dw3f = jnp.einsum('icdhw,c->idhw', dw3, wov)[:, None]
    db3f = jnp.dot(wov, db3) + bo[0]
    wd3 = _dec_w(dw3f)
    w27 = jnp.zeros((3, 3, 3, 128, 8), jnp.float32)
    for ph, (p0, p1, p2) in enumerate(_TAPS8):
        for sh, (t0, t1, t2) in enumerate(_TAPS8):
            w27 = w27.at[p0 + t0, p1 + t1, p2 + t2, :, ph].set(
                wd3[7 - ph, 7 - sh, :, 0])
    w27 = w27.reshape(27, 128, 8).astype(_BF)
    r2p = jnp.pad(r2, ((0, 0), (1, 1), (1, 1), (1, 1), (0, 0)))
    b27 = jnp.full((1, 8), db3f, jnp.float32)
    y3 = _dconv3(r2p, w27, b27)
    y3 = y3.reshape(n, 16, 16, 16, 2, 2, 2)
    y3 = y3.transpose(0, 1, 4, 2, 5, 3, 6).reshape(n, 32, 32, 32)
    r_out = y3[:, None]

    return (eq_loss, r_out)


# speedup vs baseline: 1.0251x; 1.0251x over previous
import functools
import jax
import jax.numpy as jnp
from jax import lax
from jax.experimental import pallas as pl
from jax.experimental.pallas import tpu as pltpu
from jax.experimental.pallas import tpu_sc as plsc

_TAPS8 = [(s0, s1, s2) for s0 in (0, 1) for s1 in (0, 1) for s2 in (0, 1)]
_BF = jnp.bfloat16


def _s2d(xp):
    n, d, h, w, c = xp.shape
    x = xp.reshape(n, d // 2, 2, h // 2, 2, w // 2, 2, c)
    x = x.transpose(0, 1, 3, 5, 2, 4, 6, 7)
    return x.reshape(n, d // 2, h // 2, w // 2, 8 * c)


def _enc_w(w):
    o, c = w.shape[:2]
    wr = w.astype(_BF).reshape(o, c, 2, 2, 2, 2, 2, 2)
    wr = wr.transpose(2, 4, 6, 3, 5, 7, 1, 0)
    return wr.reshape(8, 8 * c, o)


def _dec_w(w):
    c, o = w.shape[:2]
    wf = w.reshape(c, o, 2, 2, 2, 2, 2, 2)
    wr = wf.transpose(3, 5, 7, 2, 4, 6, 0, 1)
    return wr.reshape(8, 8, c, o)


def _win8(x, o):
    views = [x[:, s0:s0 + o, s1:s1 + o, s2:s2 + o, :] for s0, s1, s2 in _TAPS8]
    return jnp.stack(views, axis=4)


def _interleave(y, n, g, c):
    y = y.reshape(2, 2, 2, n, g, g, g, c)
    y = y.transpose(3, 4, 0, 5, 1, 6, 2, 7)
    return y.reshape(n, 2 * g, 2 * g, 2 * g, c)


def _conv1_body(x_ref, w_ref, b_ref, o_ref, *, g):
    x = x_ref[0]
    c = x.shape[-1]
    acc = None
    for t, (s0, s1, s2) in enumerate(_TAPS8):
        a = x[s0:s0 + g, s1:s1 + g, s2:s2 + g, :].reshape(g ** 3, c)
        p = jnp.dot(a, w_ref[t], preferred_element_type=jnp.float32)
        acc = p if acc is None else acc + p
    o_ref[0] = jnp.maximum(acc + b_ref[...], 0.0).astype(o_ref.dtype)


def _conv1(x, w, b2d, out_dtype):
    n, gp = x.shape[0], x.shape[1]
    g = gp - 1
    c = x.shape[-1]
    o = w.shape[2]
    return pl.pallas_call(
        functools.partial(_conv1_body, g=g),
        grid=(n,),
        in_specs=[
            pl.BlockSpec((1, gp, gp, gp, c), lambda i: (i, 0, 0, 0, 0)),
            pl.BlockSpec((8, c, o), lambda i: (0, 0, 0)),
            pl.BlockSpec((1, o), lambda i: (0, 0)),
        ],
        out_specs=pl.BlockSpec((1, g ** 3, o), lambda i: (i, 0, 0)),
        out_shape=jax.ShapeDtypeStruct((n, g ** 3, o), out_dtype),
    )(x, w, b2d)


def _sconv_body(x_ref, w_ref, b_ref, o_ref, *, g, relu, nb):
    x = x_ref[...]
    c = x.shape[-1]
    acc = None
    for t, (s0, s1, s2) in enumerate(_TAPS8):
        a = x[:, s0:s0 + g, s1:s1 + g, s2:s2 + g, :].reshape(nb * g ** 3, c)
        p = jnp.dot(a, w_ref[t], preferred_element_type=jnp.float32)
        acc = p if acc is None else acc + p
    acc = acc + b_ref[...]
    if relu:
        acc = jnp.maximum(acc, 0.0)
    o_ref[...] = acc.astype(o_ref.dtype)


def _sconv(x, w, b2d, relu, out_dtype, nb=2):
    n, gp = x.shape[0], x.shape[1]
    g = gp - 1
    c, o = w.shape[1], w.shape[2]
    return pl.pallas_call(
        functools.partial(_sconv_body, g=g, relu=relu, nb=nb),
        grid=(n // nb,),
        in_specs=[
            pl.BlockSpec((nb, gp, gp, gp, c), lambda i: (i, 0, 0, 0, 0)),
            pl.BlockSpec((8, c, o), lambda i: (0, 0, 0)),
            pl.BlockSpec((1, o), lambda i: (0, 0)),
        ],
        out_specs=pl.BlockSpec((nb * g ** 3, o), lambda i: (i, 0)),
        out_shape=jax.ShapeDtypeStruct((n * g ** 3, o), out_dtype),
    )(x, w, b2d)


def _dconvp_body(x_ref, w_ref, b_ref, o_ref, *, g, relu, nb):
    x = x_ref[...]
    c = x.shape[-1]
    for r, (r0, r1, r2) in enumerate(_TAPS8):
        acc = None
        for t, (s0, s1, s2) in enumerate(_TAPS8):
            a = x[:, r0 + s0:r0 + s0 + g, r1 + s1:r1 + s1 + g,
                  r2 + s2:r2 + s2 + g, :].reshape(nb * g ** 3, c)
            p = jnp.dot(a, w_ref[7 - r, 7 - t],
                        preferred_element_type=jnp.float32)
            acc = p if acc is None else acc + p
        acc = acc + b_ref[...]
        if relu:
            acc = jnp.maximum(acc, 0.0)
        o_ref[r] = acc.astype(o_ref.dtype)


def _dconvp(x, w, b2d, relu, out_dtype, nb=2):
    n, gp = x.shape[0], x.shape[1]
    g = gp - 2
    c, o = w.shape[2], w.shape[3]
    return pl.pallas_call(
        functools.partial(_dconvp_body, g=g, relu=relu, nb=nb),
        grid=(n // nb,),
        in_specs=[
            pl.BlockSpec((nb, gp, gp, gp, c), lambda i: (i, 0, 0, 0, 0)),
            pl.BlockSpec((8, 8, c, o), lambda i: (0, 0, 0, 0)),
            pl.BlockSpec((1, o), lambda i: (0, 0)),
        ],
        out_specs=pl.BlockSpec((8, nb * g ** 3, o), lambda i: (0, i, 0)),
        out_shape=jax.ShapeDtypeStruct((8, n * g ** 3, o), out_dtype),
    )(x, w, b2d)


def _dconv3_body(x_ref, w_ref, b_ref, o_ref, *, g):
    x = x_ref[0]
    c = x.shape[-1]
    acc = None
    i = 0
    for a0 in range(3):
        for a1 in range(3):
            for a2 in range(3):
                a = x[a0:a0 + g, a1:a1 + g, a2:a2 + g, :].reshape(g ** 3, c)
                p = jnp.dot(a, w_ref[i], preferred_element_type=jnp.float32)
                acc = p if acc is None else acc + p
                i += 1
    o_ref[0] = acc + b_ref[...]


def _dconv3(x, w27, b2d):
    n, gp = x.shape[0], x.shape[1]
    g = gp - 2
    c = x.shape[-1]
    return pl.pallas_call(
        functools.partial(_dconv3_body, g=g),
        grid=(n,),
        in_specs=[
            pl.BlockSpec((1, gp, gp, gp, c), lambda i: (i, 0, 0, 0, 0)),
            pl.BlockSpec((27, c, 8), lambda i: (0, 0, 0)),
            pl.BlockSpec((1, 8), lambda i: (0, 0)),
        ],
        out_specs=pl.BlockSpec((1, g * g * g, 8), lambda i: (i, 0, 0)),
        out_shape=jax.ShapeDtypeStruct((n, g * g * g, 8), jnp.float32),
    )(x, w27, b2d)


def _vq_body(flat_ref, emb_ref, idx_ref, loss_ref):
    flat = flat_ref[...]
    emb = emb_ref[...]
    f2 = jnp.sum(flat * flat, axis=1, keepdims=True)
    e2 = jnp.sum(emb * emb, axis=1)
    xe = jnp.dot(flat, emb.T, preferred_element_type=jnp.float32)
    dist = f2 + e2[None, :] - 2.0 * xe
    minv = jnp.min(dist, axis=1, keepdims=True)
    k = emb.shape[0]
    iota = lax.broadcasted_iota(jnp.int32, dist.shape, 1)
    idx_ref[...] = jnp.min(jnp.where(dist <= minv, iota, k), axis=1)
    loss_ref[...] = jnp.reshape(
        1.25 * jnp.sum(jnp.maximum(minv, 0.0)) / flat.size, (1, 1))


def _vq_assign(flat, emb):
    m, _ = flat.shape
    idx, loss = pl.pallas_call(
        _vq_body,
        out_shape=[
            jax.ShapeDtypeStruct((m,), jnp.int32),
            jax.ShapeDtypeStruct((1, 1), jnp.float32),
        ],
    )(flat, emb)
    return idx, loss[0, 0]


def _sc_gather(emb, idx):
    info = plsc.get_sparse_core_info()
    nw = info.num_cores * info.num_subcores
    b = idx.shape[0]
    d = emb.shape[1]
    bpw = b // nw
    nc = info.num_cores
    mesh = plsc.VectorSubcoreMesh(core_axis_name="c", subcore_axis_name="s")

    @functools.partial(
        pl.kernel, mesh=mesh,
        out_type=jax.ShapeDtypeStruct((b, d), jnp.float32),
        scratch_types=[
            pltpu.VMEM((bpw,), jnp.int32),
            pltpu.VMEM((bpw, d), jnp.float32),
            pltpu.SemaphoreType.DMA,
        ],
    )
    def k(emb_hbm, idx_hbm, out_hbm, idx_v, rows_v, sem):
        wid = lax.axis_index("s") * nc + lax.axis_index("c")
        base = wid * bpw
        pltpu.sync_copy(idx_hbm.at[pl.ds(base, bpw)], idx_v)
        pltpu.async_copy(emb_hbm.at[idx_v], rows_v, sem).wait()
        pltpu.sync_copy(rows_v, out_hbm.at[pl.ds(base, bpw)])

    return k(emb, idx)


def kernel(x, emb, w1, b1, w2, b2, w3, b3, dw1, db1, dw2, db2, dw3, db3, wo, bo):
    n = x.shape[0]

    xl = x.reshape(n, 32, 32, 32, 1).astype(_BF)
    xp = jnp.pad(xl, ((0, 0), (1, 1), (1, 1), (1, 1), (0, 0)))
    x2 = _s2d(xp)
    h1 = _conv1(x2, _enc_w(w1), b1.reshape(1, 128), _BF)
    h1 = h1.reshape(n, 16, 16, 16, 128)

    h1p = jnp.pad(h1, ((0, 0), (1, 1), (1, 1), (1, 1), (0, 0)))
    h1s = _s2d(h1p)
    h2 = _sconv(h1s, _enc_w(w2), b2.reshape(1, 256), True, _BF)
    h2 = h2.reshape(n, 8, 8, 8, 256)

    h2p = jnp.pad(h2, ((0, 0), (1, 1), (1, 1), (1, 1), (0, 0)))
    h2s = _s2d(h2p)
    h3 = _sconv(h2s, _enc_w(w3), b3.reshape(1, 256), False, jnp.float32)

    flat = h3.reshape(-1, emb.shape[1])
    idx, eq_loss = _vq_assign(flat, emb)
    quant = _sc_gather(emb, idx)

    q5 = quant.astype(_BF).reshape(n, 4, 4, 4, 256)
    qp = jnp.pad(q5, ((0, 0), (1, 1), (1, 1), (1, 1), (0, 0)))
    y1 = _dconvp(qp, _dec_w(dw1).astype(_BF), db1.reshape(1, 256),
                 True, _BF)
    r1 = _interleave(y1, n, 4, 256)

    r1p = jnp.pad(r1, ((0, 0), (1, 1), (1, 1), (1, 1), (0, 0)))
    y2 = _dconvp(r1p, _dec_w(dw2).astype(_BF), db2.reshape(1, 128),
                 True, _BF)
    r2 = _interleave(y2, n, 8, 128)

    wov = wo[0, :, 0, 0, 0]
    dw3f = jnp.einsum('icdhw,c->idhw', dw3, wov)[:, None]
    db3f = jnp.dot(wov, db3) + bo[0]
    wd3 = _dec_w(dw3f)
    w27 = jnp.zeros((3, 3, 3, 128, 8), jnp.float32)
    for ph, (p0, p1, p2) in enumerate(_TAPS8):
        for sh, (t0, t1, t2) in enumerate(_TAPS8):
            w27 = w27.at[p0 + t0, p1 + t1, p2 + t2, :, ph].set(
                wd3[7 - ph, 7 - sh, :, 0])
    w27 = w27.reshape(27, 128, 8).astype(_BF)
    r2p = jnp.pad(r2, ((0, 0), (1, 1), (1, 1), (1, 1), (0, 0)))
    b27 = jnp.full((1, 8), db3f, jnp.float32)
    y3 = _dconv3(r2p, w27, b27)
    y3 = y3.reshape(n, 16, 16, 16, 2, 2, 2)
    y3 = y3.transpose(0, 1, 4, 2, 5, 3, 6).reshape(n, 32, 32, 32)
    r_out = y3[:, None]

    return (eq_loss, r_out)
